# baseline (speedup 1.0000x reference)
import functools

import jax
import jax.numpy as jnp
from jax import lax
from jax.experimental import pallas as pl
from jax.experimental.pallas import tpu as pltpu
from jax.experimental.pallas import tpu_sc as plsc

EPS = 1e-5

_NC = 2
_NS = 16
_CHUNK = 128


def _onehot(nt_ref, t):
    bn = nt_ref.shape[0]
    ids = lax.broadcasted_iota(jnp.int32, (bn, t), 1)
    return (nt_ref[...] == ids).astype(jnp.float32)


def _ln_act(xb, oh, g_ref, b_ref):
    mu = jnp.mean(xb, axis=-1, keepdims=True)
    var = jnp.mean((xb - mu) ** 2, axis=-1, keepdims=True)
    h = (xb - mu) * lax.rsqrt(var + EPS)
    g = jnp.dot(oh, g_ref[...], preferred_element_type=jnp.float32)
    b = jnp.dot(oh, b_ref[...], preferred_element_type=jnp.float32)
    return jax.nn.relu(h * g + b)


def _stage_a_body(x_ref, nt_ref, g1_ref, b1_ref, w_ref, out_ref):
    d = x_ref.shape[1]
    r = w_ref.shape[1] // d
    oh = _onehot(nt_ref, g1_ref.shape[0])
    y = _ln_act(x_ref[...], oh, g1_ref, b1_ref)
    hs = jnp.dot(y, w_ref[...], preferred_element_type=jnp.float32)
    for i in range(r):
        out_ref[i] = hs[:, i * d:(i + 1) * d]


def _stage_c_body(x_ref, a_ref, nt_ref, g2_ref, b2_ref, wm_ref, bm_ref, out_ref):
    d = x_ref.shape[1]
    t = wm_ref.shape[1] // d
    x2 = x_ref[...] + a_ref[0] + a_ref[1]
    oh = _onehot(nt_ref, t)
    y = _ln_act(x2, oh, g2_ref, b2_ref)
    z = jnp.dot(y, wm_ref[...], preferred_element_type=jnp.float32)
    acc = x2 + jnp.dot(oh, bm_ref[...], preferred_element_type=jnp.float32)
    for i in range(t):
        acc = acc + oh[:, i][:, None] * z[:, i * d:(i + 1) * d]
    out_ref[...] = acc


def _make_sc_scatter(npad, ec, ch0, ch1, d):
    rows_per = npad // _NS
    b1 = _NS * ch0
    mesh = plsc.VectorSubcoreMesh(core_axis_name="c", subcore_axis_name="s")

    @functools.partial(
        pl.kernel,
        mesh=mesh,
        out_type=jax.ShapeDtypeStruct((_NC, npad, d), jnp.float32),
        scratch_types=[
            pltpu.VMEM((2, _CHUNK), jnp.int32),
            pltpu.VMEM((2, _CHUNK), jnp.int32),
            pltpu.VMEM((2, _CHUNK, d), jnp.float32),
            pltpu.VMEM_SHARED((npad, d), jnp.float32),
        ] + [pltpu.SemaphoreType.DMA] * 4,
    )
    def sc_scatter(hs, gidx, didx, zeros, out, dv, gq, rows, acc, *sems):
        c = lax.axis_index("c")
        s = lax.axis_index("s")
        base = lax.select(c == 0, s * ch0, b1 + s * ch1)
        chc = lax.select(c == 0, jnp.int32(ch0), jnp.int32(ch1))
        nch = lax.max(0, lax.min(chc, ec - base))
        isems = sems[0:2]
        gsems = sems[2:4]
        pltpu.sync_copy(zeros, acc.at[pl.ds(s * rows_per, rows_per)])
        plsc.subcore_barrier()

        def load_idx(j, b):
            pltpu.async_copy(gidx.at[base + j], gq.at[b], isems[b])
            pltpu.async_copy(didx.at[base + j], dv.at[b], isems[b])

        def wait_idx(j, b):
            pltpu.make_async_copy(gidx.at[base + j], gq.at[b],
                                  isems[b]).wait()
            pltpu.make_async_copy(didx.at[base + j], dv.at[b],
                                  isems[b]).wait()

        def fire_gather(b):
            pltpu.async_copy(hs.at[gq.at[b]], rows.at[b], gsems[b])

        def finish_chunk(b):
            pltpu.make_async_copy(hs.at[gq.at[b]], rows.at[b],
                                  gsems[b]).wait()
            pltpu.sync_copy(rows.at[b], acc.at[dv.at[b]], add=True)

        @pl.when(nch > 0)
        def _():
            load_idx(0, 0)

            @pl.when(nch > 1)
            def _():
                load_idx(1, 1)
            wait_idx(0, 0)
            fire_gather(0)

        def body(j):
            for b in range(2):
                jj = j + b
                @pl.when(jj + 1 < nch)
                def _():
                    wait_idx(jj + 1, 1 - b)
                    fire_gather(1 - b)

                finish_chunk(b)

                @pl.when(jj + 2 < nch)
                def _():
                    load_idx(jj + 2, b)

        pl.loop(0, nch & ~1, step=2)(body)
        @pl.when(nch % 2 == 1)
        def _():
            finish_chunk(0)
        plsc.subcore_barrier()
        pltpu.sync_copy(acc.at[pl.ds(s * rows_per, rows_per)],
                        out.at[c, pl.ds(s * rows_per, rows_per)])

    return sc_scatter


def kernel(x, adj_t, node_type, edge_type, gamma1, beta1, W_conv, gamma2,
           beta2, W_mlp, b_mlp):
    n, d = x.shape
    r = W_conv.shape[0]
    t = W_mlp.shape[0]
    e = edge_type.shape[0]

    nt2 = node_type.astype(jnp.int32).reshape(n, 1)

    bn = 1000
    grid_a = n // bn
    hs = pl.pallas_call(
        _stage_a_body,
        grid=(grid_a,),
        in_specs=[
            pl.BlockSpec((bn, d), lambda i: (i, 0)),
            pl.BlockSpec((bn, 1), lambda i: (i, 0)),
            pl.BlockSpec((t, d), lambda i: (0, 0)),
            pl.BlockSpec((t, d), lambda i: (0, 0)),
            pl.BlockSpec((d, r * d), lambda i: (0, 0)),
        ],
        out_specs=pl.BlockSpec((r, bn, d), lambda i: (0, i, 0)),
        out_shape=jax.ShapeDtypeStruct((r, n, d), jnp.float32),
    )(x, nt2, gamma1, beta1,
      jnp.moveaxis(W_conv, 0, 1).reshape(d, r * d))
    hs_flat = hs.reshape(r * n, d)

    assert e % _CHUNK == 0
    ec = e // _CHUNK
    ch0 = round(0.515 * ec / _NS)
    ch1 = -(-(ec - _NS * ch0) // _NS)
    npad = -(-n // (_NS * 8)) * (_NS * 8)
    assert (npad // _NS) % 8 == 0

    a32 = adj_t.astype(jnp.int32)
    etv = edge_type.astype(jnp.int32).reshape(ec, _CHUNK)
    gidx2 = etv * n + a32[0].reshape(ec, _CHUNK)
    didx2 = a32[1].reshape(ec, _CHUNK)
    zeros = jnp.zeros((npad // _NS, d), jnp.float32)

    agg2 = _make_sc_scatter(npad, ec, ch0, ch1, d)(hs_flat, gidx2, didx2,
                                                   zeros)

    out = pl.pallas_call(
        _stage_c_body,
        grid=(grid_a,),
        in_specs=[
            pl.BlockSpec((bn, d), lambda i: (i, 0)),
            pl.BlockSpec((_NC, bn, d), lambda i: (0, i, 0)),
            pl.BlockSpec((bn, 1), lambda i: (i, 0)),
            pl.BlockSpec((t, d), lambda i: (0, 0)),
            pl.BlockSpec((t, d), lambda i: (0, 0)),
            pl.BlockSpec((d, t * d), lambda i: (0, 0)),
            pl.BlockSpec((t, d), lambda i: (0, 0)),
        ],
        out_specs=pl.BlockSpec((bn, d), lambda i: (i, 0)),
        out_shape=jax.ShapeDtypeStruct((n, d), jnp.float32),
    )(x, agg2, nt2, gamma2, beta2,
      jnp.moveaxis(W_mlp, 0, 1).reshape(d, t * d), b_mlp)
    return out


# speedup vs baseline: 1.0873x; 1.0873x over previous
import functools

import jax
import jax.numpy as jnp
from jax import lax
from jax.experimental import pallas as pl
from jax.experimental.pallas import tpu as pltpu
from jax.experimental.pallas import tpu_sc as plsc

EPS = 1e-5

_NC = 2
_NS = 16
_CHUNK = 128


def _onehot(nt_ref, t):
    bn = nt_ref.shape[0]
    ids = lax.broadcasted_iota(jnp.int32, (bn, t), 1)
    return (nt_ref[...] == ids).astype(jnp.float32)


def _ln_act(xb, oh, g_ref, b_ref):
    mu = jnp.mean(xb, axis=-1, keepdims=True)
    var = jnp.mean((xb - mu) ** 2, axis=-1, keepdims=True)
    h = (xb - mu) * lax.rsqrt(var + EPS)
    g = jnp.dot(oh, g_ref[...], preferred_element_type=jnp.float32)
    b = jnp.dot(oh, b_ref[...], preferred_element_type=jnp.float32)
    return jax.nn.relu(h * g + b)


def _stage_a_body(x_ref, nt_ref, g1_ref, b1_ref, w_ref, out_ref):
    d = x_ref.shape[1]
    r = w_ref.shape[1] // d
    oh = _onehot(nt_ref, g1_ref.shape[0])
    y = _ln_act(x_ref[...], oh, g1_ref, b1_ref)
    hs = jnp.dot(y, w_ref[...], preferred_element_type=jnp.float32)
    for i in range(r):
        out_ref[i] = hs[:, i * d:(i + 1) * d]


def _stage_c_body(x_ref, a_ref, nt_ref, g2_ref, b2_ref, wm_ref, bm_ref, out_ref):
    d = x_ref.shape[1]
    t = wm_ref.shape[1] // d
    x2 = x_ref[...] + a_ref[0] + a_ref[1]
    oh = _onehot(nt_ref, t)
    y = _ln_act(x2, oh, g2_ref, b2_ref)
    z = jnp.dot(y, wm_ref[...], preferred_element_type=jnp.float32)
    acc = x2 + jnp.dot(oh, bm_ref[...], preferred_element_type=jnp.float32)
    for i in range(t):
        acc = acc + oh[:, i][:, None] * z[:, i * d:(i + 1) * d]
    out_ref[...] = acc


def _make_sc_scatter(npad, ec, ch0, ch1, d):
    rows_per = npad // _NS
    b1 = _NS * ch0
    mesh = plsc.VectorSubcoreMesh(core_axis_name="c", subcore_axis_name="s")

    @functools.partial(
        pl.kernel,
        mesh=mesh,
        out_type=jax.ShapeDtypeStruct((_NC, npad, d), jnp.float32),
        scratch_types=[
            pltpu.VMEM((2, _CHUNK), jnp.int32),
            pltpu.VMEM((2, _CHUNK), jnp.int32),
            pltpu.VMEM((2, _CHUNK, d), jnp.float32),
            pltpu.VMEM_SHARED((npad, d), jnp.float32),
        ] + [pltpu.SemaphoreType.DMA] * 4,
    )
    def sc_scatter(hs, gidx, didx, zeros, out, dv, gq, rows, acc, *sems):
        c = lax.axis_index("c")
        s = lax.axis_index("s")
        base = lax.select(c == 0, s * ch0, b1 + s * ch1)
        chc = lax.select(c == 0, jnp.int32(ch0), jnp.int32(ch1))
        nch = lax.max(0, lax.min(chc, ec - base))
        isems = sems[0:2]
        gsems = sems[2:4]
        pltpu.sync_copy(zeros, acc.at[pl.ds(s * rows_per, rows_per)])
        plsc.subcore_barrier()

        def load_idx(j, b):
            pltpu.async_copy(gidx.at[base + j], gq.at[b], isems[b])
            pltpu.async_copy(didx.at[base + j], dv.at[b], isems[b])

        def wait_idx(j, b):
            pltpu.make_async_copy(gidx.at[base + j], gq.at[b],
                                  isems[b]).wait()
            pltpu.make_async_copy(didx.at[base + j], dv.at[b],
                                  isems[b]).wait()

        def fire_gather(b):
            pltpu.async_copy(hs.at[gq.at[b]], rows.at[b], gsems[b])

        def finish_chunk(b):
            pltpu.make_async_copy(hs.at[gq.at[b]], rows.at[b],
                                  gsems[b]).wait()
            pltpu.sync_copy(rows.at[b], acc.at[dv.at[b]], add=True)

        @pl.when(nch > 0)
        def _():
            load_idx(0, 0)

            @pl.when(nch > 1)
            def _():
                load_idx(1, 1)
            wait_idx(0, 0)
            fire_gather(0)

        def body(j):
            for b in range(2):
                jj = j + b
                @pl.when(jj + 1 < nch)
                def _():
                    wait_idx(jj + 1, 1 - b)
                    fire_gather(1 - b)

                finish_chunk(b)

                @pl.when(jj + 2 < nch)
                def _():
                    load_idx(jj + 2, b)

        pl.loop(0, nch & ~1, step=2)(body)
        @pl.when(nch % 2 == 1)
        def _():
            finish_chunk(0)
        plsc.subcore_barrier()
        pltpu.sync_copy(acc.at[pl.ds(s * rows_per, rows_per)],
                        out.at[c, pl.ds(s * rows_per, rows_per)])

    return sc_scatter


def kernel(x, adj_t, node_type, edge_type, gamma1, beta1, W_conv, gamma2,
           beta2, W_mlp, b_mlp):
    n, d = x.shape
    r = W_conv.shape[0]
    t = W_mlp.shape[0]
    e = edge_type.shape[0]

    nt2 = node_type.astype(jnp.int32).reshape(n, 1)

    bn = 2000
    grid_a = n // bn
    hs = pl.pallas_call(
        _stage_a_body,
        grid=(grid_a,),
        in_specs=[
            pl.BlockSpec((bn, d), lambda i: (i, 0)),
            pl.BlockSpec((bn, 1), lambda i: (i, 0)),
            pl.BlockSpec((t, d), lambda i: (0, 0)),
            pl.BlockSpec((t, d), lambda i: (0, 0)),
            pl.BlockSpec((d, r * d), lambda i: (0, 0)),
        ],
        out_specs=pl.BlockSpec((r, bn, d), lambda i: (0, i, 0)),
        out_shape=jax.ShapeDtypeStruct((r, n, d), jnp.float32),
    )(x, nt2, gamma1, beta1,
      jnp.moveaxis(W_conv, 0, 1).reshape(d, r * d))
    hs_flat = hs.reshape(r * n, d)

    assert e % _CHUNK == 0
    ec = e // _CHUNK
    ch0 = round(0.52 * ec / _NS)
    ch1 = -(-(ec - _NS * ch0) // _NS)
    npad = -(-n // (_NS * 8)) * (_NS * 8)
    assert (npad // _NS) % 8 == 0

    adjv = jnp.swapaxes(adj_t.astype(jnp.int32).reshape(2, ec, _CHUNK), 0, 1)
    etv = edge_type.astype(jnp.int32).reshape(ec, _CHUNK)
    gidx2 = etv * n + adjv[:, 0, :]
    didx2 = adjv[:, 1, :]
    zeros = jnp.zeros((npad // _NS, d), jnp.float32)

    agg2 = _make_sc_scatter(npad, ec, ch0, ch1, d)(hs_flat, gidx2, didx2,
                                                   zeros)

    out = pl.pallas_call(
        _stage_c_body,
        grid=(grid_a,),
        in_specs=[
            pl.BlockSpec((bn, d), lambda i: (i, 0)),
            pl.BlockSpec((_NC, bn, d), lambda i: (0, i, 0)),
            pl.BlockSpec((bn, 1), lambda i: (i, 0)),
            pl.BlockSpec((t, d), lambda i: (0, 0)),
            pl.BlockSpec((t, d), lambda i: (0, 0)),
            pl.BlockSpec((d, t * d), lambda i: (0, 0)),
            pl.BlockSpec((t, d), lambda i: (0, 0)),
        ],
        out_specs=pl.BlockSpec((bn, d), lambda i: (i, 0)),
        out_shape=jax.ShapeDtypeStruct((n, d), jnp.float32),
    )(x, agg2, nt2, gamma2, beta2,
      jnp.moveaxis(W_mlp, 0, 1).reshape(d, t * d), b_mlp)
    return out


# speedup vs baseline: 1.0921x; 1.0044x over previous
import functools

import jax
import jax.numpy as jnp
from jax import lax
from jax.experimental import pallas as pl
from jax.experimental.pallas import tpu as pltpu
from jax.experimental.pallas import tpu_sc as plsc

EPS = 1e-5

_NC = 2
_NS = 16
_CHUNK = 128


def _onehot(nt_ref, t):
    bn = nt_ref.shape[0]
    ids = lax.broadcasted_iota(jnp.int32, (bn, t), 1)
    return (nt_ref[...] == ids).astype(jnp.float32)


def _ln_act(xb, oh, g_ref, b_ref):
    mu = jnp.mean(xb, axis=-1, keepdims=True)
    var = jnp.mean((xb - mu) ** 2, axis=-1, keepdims=True)
    h = (xb - mu) * lax.rsqrt(var + EPS)
    g = jnp.dot(oh, g_ref[...], preferred_element_type=jnp.float32)
    b = jnp.dot(oh, b_ref[...], preferred_element_type=jnp.float32)
    return jax.nn.relu(h * g + b)


def _stage_a_body(x_ref, nt_ref, g1_ref, b1_ref, w_ref, out_ref):
    d = x_ref.shape[1]
    r = w_ref.shape[1] // d
    oh = _onehot(nt_ref, g1_ref.shape[0])
    y = _ln_act(x_ref[...], oh, g1_ref, b1_ref)
    hs = jnp.dot(y, w_ref[...], preferred_element_type=jnp.float32)
    for i in range(r):
        out_ref[i] = hs[:, i * d:(i + 1) * d]


def _stage_c_body(x_ref, a_ref, nt_ref, g2_ref, b2_ref, wm_ref, bm_ref, out_ref):
    d = x_ref.shape[1]
    t = wm_ref.shape[1] // d
    x2 = x_ref[...] + a_ref[0] + a_ref[1]
    oh = _onehot(nt_ref, t)
    y = _ln_act(x2, oh, g2_ref, b2_ref)
    z = jnp.dot(y, wm_ref[...], preferred_element_type=jnp.float32)
    acc = x2 + jnp.dot(oh, bm_ref[...], preferred_element_type=jnp.float32)
    for i in range(t):
        acc = acc + oh[:, i][:, None] * z[:, i * d:(i + 1) * d]
    out_ref[...] = acc


def _make_sc_scatter(npad, ec, ch0, ch1, d):
    rows_per = npad // _NS
    b1 = _NS * ch0
    mesh = plsc.VectorSubcoreMesh(core_axis_name="c", subcore_axis_name="s")

    @functools.partial(
        pl.kernel,
        mesh=mesh,
        out_type=jax.ShapeDtypeStruct((_NC, npad, d), jnp.float32),
        scratch_types=[
            pltpu.VMEM((2, _CHUNK), jnp.int32),
            pltpu.VMEM((2, _CHUNK), jnp.int32),
            pltpu.VMEM((2, _CHUNK, d), jnp.float32),
            pltpu.VMEM_SHARED((npad, d), jnp.float32),
        ] + [pltpu.SemaphoreType.DMA] * 4,
    )
    def sc_scatter(hs, gidx, didx, zeros, out, dv, gq, rows, acc, *sems):
        c = lax.axis_index("c")
        s = lax.axis_index("s")
        base = lax.select(c == 0, s * ch0, b1 + s * ch1)
        chc = lax.select(c == 0, jnp.int32(ch0), jnp.int32(ch1))
        nch = lax.max(0, lax.min(chc, ec - base))
        isems = sems[0:2]
        gsems = sems[2:4]
        pltpu.sync_copy(zeros, acc.at[pl.ds(s * rows_per, rows_per)])
        plsc.subcore_barrier()

        def load_idx(j, b):
            pltpu.async_copy(gidx.at[base + j], gq.at[b], isems[b])
            pltpu.async_copy(didx.at[base + j], dv.at[b], isems[b])

        def wait_idx(j, b):
            pltpu.make_async_copy(gidx.at[base + j], gq.at[b],
                                  isems[b]).wait()
            pltpu.make_async_copy(didx.at[base + j], dv.at[b],
                                  isems[b]).wait()

        def fire_gather(b):
            pltpu.async_copy(hs.at[gq.at[b]], rows.at[b], gsems[b])

        def finish_chunk(b):
            pltpu.make_async_copy(hs.at[gq.at[b]], rows.at[b],
                                  gsems[b]).wait()
            pltpu.sync_copy(rows.at[b], acc.at[dv.at[b]], add=True)

        @pl.when(nch > 0)
        def _():
            load_idx(0, 0)

            @pl.when(nch > 1)
            def _():
                load_idx(1, 1)
            wait_idx(0, 0)
            fire_gather(0)

        def body(j):
            for b in range(2):
                jj = j + b
                @pl.when(jj + 1 < nch)
                def _():
                    wait_idx(jj + 1, 1 - b)
                    fire_gather(1 - b)

                finish_chunk(b)

                @pl.when(jj + 2 < nch)
                def _():
                    load_idx(jj + 2, b)

        pl.loop(0, nch & ~1, step=2)(body)
        @pl.when(nch % 2 == 1)
        def _():
            finish_chunk(0)
        plsc.subcore_barrier()
        pltpu.sync_copy(acc.at[pl.ds(s * rows_per, rows_per)],
                        out.at[c, pl.ds(s * rows_per, rows_per)])

    return sc_scatter


def kernel(x, adj_t, node_type, edge_type, gamma1, beta1, W_conv, gamma2,
           beta2, W_mlp, b_mlp):
    n, d = x.shape
    r = W_conv.shape[0]
    t = W_mlp.shape[0]
    e = edge_type.shape[0]

    nt2 = node_type.astype(jnp.int32).reshape(n, 1)

    bn = 2000
    grid_a = n // bn
    hs = pl.pallas_call(
        _stage_a_body,
        grid=(grid_a,),
        in_specs=[
            pl.BlockSpec((bn, d), lambda i: (i, 0)),
            pl.BlockSpec((bn, 1), lambda i: (i, 0)),
            pl.BlockSpec((t, d), lambda i: (0, 0)),
            pl.BlockSpec((t, d), lambda i: (0, 0)),
            pl.BlockSpec((d, r * d), lambda i: (0, 0)),
        ],
        out_specs=pl.BlockSpec((r, bn, d), lambda i: (0, i, 0)),
        out_shape=jax.ShapeDtypeStruct((r, n, d), jnp.float32),
    )(x, nt2, gamma1, beta1,
      jnp.moveaxis(W_conv, 0, 1).reshape(d, r * d))
    hs_flat = hs.reshape(r * n, d)

    assert e % _CHUNK == 0
    ec = e // _CHUNK
    ch0 = round(0.51 * ec / _NS)
    ch1 = -(-(ec - _NS * ch0) // _NS)
    npad = -(-n // (_NS * 8)) * (_NS * 8)
    assert (npad // _NS) % 8 == 0

    adjv = jnp.swapaxes(adj_t.astype(jnp.int32).reshape(2, ec, _CHUNK), 0, 1)
    etv = edge_type.astype(jnp.int32).reshape(ec, _CHUNK)
    gidx2 = etv * n + adjv[:, 0, :]
    didx2 = adjv[:, 1, :]
    zeros = jnp.zeros((npad // _NS, d), jnp.float32)

    agg2 = _make_sc_scatter(npad, ec, ch0, ch1, d)(hs_flat, gidx2, didx2,
                                                   zeros)

    out = pl.pallas_call(
        _stage_c_body,
        grid=(grid_a,),
        in_specs=[
            pl.BlockSpec((bn, d), lambda i: (i, 0)),
            pl.BlockSpec((_NC, bn, d), lambda i: (0, i, 0)),
            pl.BlockSpec((bn, 1), lambda i: (i, 0)),
            pl.BlockSpec((t, d), lambda i: (0, 0)),
            pl.BlockSpec((t, d), lambda i: (0, 0)),
            pl.BlockSpec((d, t * d), lambda i: (0, 0)),
            pl.BlockSpec((t, d), lambda i: (0, 0)),
        ],
        out_specs=pl.BlockSpec((bn, d), lambda i: (i, 0)),
        out_shape=jax.ShapeDtypeStruct((n, d), jnp.float32),
    )(x, agg2, nt2, gamma2, beta2,
      jnp.moveaxis(W_mlp, 0, 1).reshape(d, t * d), b_mlp)
    return out
